# baseline (device time: 114777 ns/iter reference)
import jax
import jax.numpy as jnp
from jax import lax
from jax.experimental import pallas as pl
from jax.experimental.pallas import tpu as pltpu

T = 1024
D = 2048
V_SHARD = 16384
BN = 2048
NJ = V_SHARD // BN


def kernel(x, W, labels):
    labels2d = labels.reshape(T, 1)

    def body(x_ref, w_ref, lab_ref, out_ref,
             s_ref, gl_ref, send_ref, recv_ref, send_sem, recv_sem):
        j = pl.program_id(0)
        my_x = lax.axis_index("x")
        my_y = lax.axis_index("y")
        my_z = lax.axis_index("z")
        partner = (my_x, 1 - my_y, my_z)

        @pl.when(j == 0)
        def _init():
            s_ref[...] = jnp.zeros((T, 1), jnp.float32)
            gl_ref[...] = jnp.zeros((T, 1), jnp.float32)
            barrier = pltpu.get_barrier_semaphore()
            pl.semaphore_signal(
                barrier, inc=1,
                device_id=partner,
                device_id_type=pl.DeviceIdType.MESH,
            )
            pl.semaphore_wait(barrier, 1)

        logits = jnp.dot(x_ref[...], w_ref[...],
                         preferred_element_type=jnp.float32)
        e = jnp.exp(logits)

        local_label = lab_ref[...] - my_y * V_SHARD
        col_ids = lax.broadcasted_iota(jnp.int32, (T, BN), 1)
        hit = col_ids == (local_label - j * BN)
        masked = jnp.where(hit, logits, 0.0)

        ones_mat = jnp.ones((BN, 128), jnp.float32)
        s_ref[...] += jnp.dot(e, ones_mat,
                              preferred_element_type=jnp.float32)[:, 0:1]
        gl_ref[...] += jnp.dot(masked, ones_mat,
                               preferred_element_type=jnp.float32)[:, 0:1]

        @pl.when(j == NJ - 1)
        def _finish():
            lse_local = jnp.log(s_ref[...])
            send_ref[:, 0:1] = lse_local
            send_ref[:, 1:2] = gl_ref[...]
            rdma = pltpu.make_async_remote_copy(
                src_ref=send_ref,
                dst_ref=recv_ref,
                send_sem=send_sem,
                recv_sem=recv_sem,
                device_id=partner,
                device_id_type=pl.DeviceIdType.MESH,
            )
            rdma.start()
            rdma.wait()
            lse_other = recv_ref[:, 0:1]
            gl_other = recv_ref[:, 1:2]
            mx = jnp.maximum(lse_local, lse_other)
            lse = mx + jnp.log(jnp.exp(lse_local - mx)
                               + jnp.exp(lse_other - mx))
            out_ref[...] = lse - (gl_ref[...] + gl_other)

    out = pl.pallas_call(
        body,
        grid=(NJ,),
        in_specs=[
            pl.BlockSpec((T, D), lambda j: (0, 0)),
            pl.BlockSpec((D, BN), lambda j: (0, j)),
            pl.BlockSpec((T, 1), lambda j: (0, 0)),
        ],
        out_specs=pl.BlockSpec((T, 1), lambda j: (0, 0)),
        out_shape=jax.ShapeDtypeStruct((T, 1), jnp.float32),
        scratch_shapes=[
            pltpu.VMEM((T, 1), jnp.float32),
            pltpu.VMEM((T, 1), jnp.float32),
            pltpu.VMEM((T, 2), jnp.float32),
            pltpu.VMEM((T, 2), jnp.float32),
            pltpu.SemaphoreType.DMA,
            pltpu.SemaphoreType.DMA,
        ],
        compiler_params=pltpu.CompilerParams(
            dimension_semantics=("arbitrary",),
            collective_id=0,
            vmem_limit_bytes=64 * 1024 * 1024,
        ),
    )(x, W, labels2d)
    return out.reshape(T)


# device time: 99690 ns/iter; 1.1513x vs baseline; 1.1513x over previous
import jax
import jax.numpy as jnp
from jax import lax
from jax.experimental import pallas as pl
from jax.experimental.pallas import tpu as pltpu

T = 1024
D = 2048
V_SHARD = 16384
BN = 2048
NJ = V_SHARD // BN


def kernel(x, W, labels):
    labels2d = labels.reshape(T, 1)

    def body(x_ref, w_ref, lab_ref, out_ref,
             s_ref, gl_ref, send_ref, recv_ref, send_sem, recv_sem):
        j = pl.program_id(0)
        my_x = lax.axis_index("x")
        my_y = lax.axis_index("y")
        my_z = lax.axis_index("z")
        partner = (my_x, 1 - my_y, my_z)

        @pl.when(j == 0)
        def _init():
            s_ref[...] = jnp.zeros((T, 1), jnp.float32)
            gl_ref[...] = jnp.zeros((T, 1), jnp.float32)
            barrier = pltpu.get_barrier_semaphore()
            pl.semaphore_signal(
                barrier, inc=1,
                device_id=partner,
                device_id_type=pl.DeviceIdType.MESH,
            )
            pl.semaphore_wait(barrier, 1)

        logits = jnp.dot(x_ref[...], w_ref[...],
                         preferred_element_type=jnp.float32,
                         precision=lax.Precision.DEFAULT)
        s_ref[...] += jnp.sum(jnp.exp(logits), axis=1, keepdims=True)

        local_label = lab_ref[...] - my_y * V_SHARD
        col_ids = lax.broadcasted_iota(jnp.int32, (T, BN), 1)
        hit = col_ids == (local_label - j * BN)
        gl_ref[...] += jnp.sum(jnp.where(hit, logits, 0.0), axis=1,
                               keepdims=True)

        @pl.when(j == NJ - 1)
        def _finish():
            lse_local = jnp.log(s_ref[...])
            send_ref[:, 0:1] = lse_local
            send_ref[:, 1:2] = gl_ref[...]
            rdma = pltpu.make_async_remote_copy(
                src_ref=send_ref,
                dst_ref=recv_ref,
                send_sem=send_sem,
                recv_sem=recv_sem,
                device_id=partner,
                device_id_type=pl.DeviceIdType.MESH,
            )
            rdma.start()
            rdma.wait()
            lse_other = recv_ref[:, 0:1]
            gl_other = recv_ref[:, 1:2]
            mx = jnp.maximum(lse_local, lse_other)
            lse = mx + jnp.log(jnp.exp(lse_local - mx)
                               + jnp.exp(lse_other - mx))
            out_ref[...] = lse - (gl_ref[...] + gl_other)

    out = pl.pallas_call(
        body,
        grid=(NJ,),
        in_specs=[
            pl.BlockSpec((T, D), lambda j: (0, 0)),
            pl.BlockSpec((D, BN), lambda j: (0, j)),
            pl.BlockSpec((T, 1), lambda j: (0, 0)),
        ],
        out_specs=pl.BlockSpec((T, 1), lambda j: (0, 0)),
        out_shape=jax.ShapeDtypeStruct((T, 1), jnp.float32),
        scratch_shapes=[
            pltpu.VMEM((T, 1), jnp.float32),
            pltpu.VMEM((T, 1), jnp.float32),
            pltpu.VMEM((T, 2), jnp.float32),
            pltpu.VMEM((T, 2), jnp.float32),
            pltpu.SemaphoreType.DMA,
            pltpu.SemaphoreType.DMA,
        ],
        compiler_params=pltpu.CompilerParams(
            dimension_semantics=("arbitrary",),
            collective_id=0,
            vmem_limit_bytes=64 * 1024 * 1024,
        ),
    )(x, W, labels2d)
    return out.reshape(T)


# device time: 98695 ns/iter; 1.1629x vs baseline; 1.0101x over previous
import jax
import jax.numpy as jnp
from jax import lax
from jax.experimental import pallas as pl
from jax.experimental.pallas import tpu as pltpu

T = 1024
D = 2048
V_SHARD = 16384
BN = 2048
NJ = V_SHARD // BN


def kernel(x, W, labels):
    labels2d = labels.reshape(T, 1)

    def body(x_ref, w_ref, lab_ref, out_ref,
             xb_ref, s_ref, gl_ref, send_ref, recv_ref, send_sem, recv_sem):
        j = pl.program_id(0)
        my_x = lax.axis_index("x")
        my_y = lax.axis_index("y")
        my_z = lax.axis_index("z")
        partner = (my_x, 1 - my_y, my_z)

        @pl.when(j == 0)
        def _init():
            s_ref[...] = jnp.zeros((T, 1), jnp.float32)
            gl_ref[...] = jnp.zeros((T, 1), jnp.float32)
            xb_ref[...] = x_ref[...].astype(jnp.bfloat16)
            barrier = pltpu.get_barrier_semaphore()
            pl.semaphore_signal(
                barrier, inc=1,
                device_id=partner,
                device_id_type=pl.DeviceIdType.MESH,
            )
            pl.semaphore_wait(barrier, 1)

        logits = jnp.dot(xb_ref[...], w_ref[...].astype(jnp.bfloat16),
                         preferred_element_type=jnp.float32)
        s_ref[...] += jnp.sum(jnp.exp(logits), axis=1, keepdims=True)

        local_label = lab_ref[...] - my_y * V_SHARD
        col_ids = lax.broadcasted_iota(jnp.int32, (T, BN), 1)
        hit = col_ids == (local_label - j * BN)
        gl_ref[...] += jnp.sum(jnp.where(hit, logits, 0.0), axis=1,
                               keepdims=True)

        @pl.when(j == NJ - 1)
        def _finish():
            lse_local = jnp.log(s_ref[...])
            send_ref[:, 0:1] = lse_local
            send_ref[:, 1:2] = gl_ref[...]
            rdma = pltpu.make_async_remote_copy(
                src_ref=send_ref,
                dst_ref=recv_ref,
                send_sem=send_sem,
                recv_sem=recv_sem,
                device_id=partner,
                device_id_type=pl.DeviceIdType.MESH,
            )
            rdma.start()
            rdma.wait()
            lse_other = recv_ref[:, 0:1]
            gl_other = recv_ref[:, 1:2]
            mx = jnp.maximum(lse_local, lse_other)
            lse = mx + jnp.log(jnp.exp(lse_local - mx)
                               + jnp.exp(lse_other - mx))
            out_ref[...] = lse - (gl_ref[...] + gl_other)

    out = pl.pallas_call(
        body,
        grid=(NJ,),
        in_specs=[
            pl.BlockSpec((T, D), lambda j: (0, 0)),
            pl.BlockSpec((D, BN), lambda j: (0, j)),
            pl.BlockSpec((T, 1), lambda j: (0, 0)),
        ],
        out_specs=pl.BlockSpec((T, 1), lambda j: (0, 0)),
        out_shape=jax.ShapeDtypeStruct((T, 1), jnp.float32),
        scratch_shapes=[
            pltpu.VMEM((T, D), jnp.bfloat16),
            pltpu.VMEM((T, 1), jnp.float32),
            pltpu.VMEM((T, 1), jnp.float32),
            pltpu.VMEM((T, 2), jnp.float32),
            pltpu.VMEM((T, 2), jnp.float32),
            pltpu.SemaphoreType.DMA,
            pltpu.SemaphoreType.DMA,
        ],
        compiler_params=pltpu.CompilerParams(
            dimension_semantics=("arbitrary",),
            collective_id=0,
            vmem_limit_bytes=64 * 1024 * 1024,
        ),
    )(x, W, labels2d)
    return out.reshape(T)
